# Initial kernel scaffold; baseline (speedup 1.0000x reference)
import jax
import jax.numpy as jnp
from jax.experimental import pallas as pl


def kernel(input_ids, emb_table, W, b):
    raise NotImplementedError("write your pallas kernel here")


# speedup vs baseline: 5.1131x; 5.1131x over previous
import functools

import jax
import jax.numpy as jnp
from jax import lax
from jax.experimental import pallas as pl
from jax.experimental.pallas import tpu as pltpu
from jax.experimental.pallas import tpu_sc as plsc

NC, NS = 2, 16
NW = NC * NS
LANE = 16
VPAD = 128


def _fold_linear(emb_pad, W, b):
    O = W.shape[0]

    def body(emb_ref, w_ref, b_ref, t_ref):
        t_ref[...] = lax.dot_general(
            w_ref[...], emb_ref[...], (((1,), (1,)), ((), ())),
            preferred_element_type=jnp.float32) + b_ref[...]

    return pl.pallas_call(
        body,
        out_shape=jax.ShapeDtypeStruct((O, VPAD), jnp.float32),
    )(emb_pad, W, b.reshape(O, 1))


def _sc_lookup(table4, ids_flat, n_ids):
    O = table4.shape[0]
    ids_per_w = n_ids // NW
    K = 2048
    n_chunks = ids_per_w // K
    n_sub = K // LANE
    mesh = plsc.VectorSubcoreMesh(core_axis_name="c", subcore_axis_name="s")

    @functools.partial(
        pl.kernel,
        out_type=jax.ShapeDtypeStruct((n_ids * O,), jnp.float32),
        mesh=mesh,
        scratch_types=[
            pltpu.VMEM((O, VPAD), jnp.float32),
            pltpu.VMEM((K,), jnp.int32),
            pltpu.VMEM((K * O,), jnp.float32),
        ],
        compiler_params=pltpu.CompilerParams(
            use_tc_tiling_on_sc=False, needs_layout_passes=False),
    )
    def body(t_ref, ids_ref, out_ref, t_v, idx_v, out_v):
        wid = lax.axis_index("s") * NC + lax.axis_index("c")
        pltpu.sync_copy(t_ref, t_v)
        iota = lax.iota(jnp.int32, LANE)
        col_splat = [jnp.full((LANE,), c, jnp.int32) for c in range(O)]
        scat = [iota * O + c for c in range(O)]

        def chunk_fn(i, carry):
            base = (wid * n_chunks + i) * K
            pltpu.sync_copy(ids_ref.at[pl.ds(base, K)], idx_v)

            def sub_fn(s, carry2):
                ids16 = idx_v[pl.ds(s * LANE, LANE)]
                dst = s * (LANE * O)
                for c in range(O):
                    vals = plsc.load_gather(t_v, [col_splat[c], ids16])
                    plsc.store_scatter(out_v, [scat[c] + dst], vals)
                return carry2

            lax.fori_loop(0, n_sub, sub_fn, 0, unroll=8)
            pltpu.sync_copy(out_v, out_ref.at[pl.ds(base * O, K * O)])
            return carry

        lax.fori_loop(0, n_chunks, chunk_fn, 0)

    return body(table4, ids_flat)


def kernel(input_ids, emb_table, W, b):
    Bsz, Lseq = input_ids.shape
    V, D = emb_table.shape
    O = W.shape[0]
    emb_pad = jnp.zeros((VPAD, D), jnp.float32).at[:V].set(emb_table)
    table4 = _fold_linear(emb_pad, W, b)
    n_ids = Bsz * Lseq
    out_flat = _sc_lookup(table4, input_ids.reshape(n_ids), n_ids)
    return out_flat.reshape(Bsz, Lseq, O)


# speedup vs baseline: 60.6890x; 11.8693x over previous
import functools

import jax
import jax.numpy as jnp
from jax import lax
from jax.experimental import pallas as pl
from jax.experimental.pallas import tpu as pltpu
from jax.experimental.pallas import tpu_sc as plsc

NC, NS = 2, 16
NW = NC * NS
LANE = 16
VPAD = 128

CB = 25
RB = 128
GRP = 1024
CCH = 5
RPW = RB // NW
ITERS = RPW * (CB // CCH)


def _fold_linear(emb_pad, W, b):
    O = W.shape[0]

    def body(emb_ref, w_ref, b_ref, t_ref):
        t_ref[...] = lax.dot_general(
            w_ref[...], emb_ref[...], (((1,), (1,)), ((), ())),
            preferred_element_type=jnp.float32) + b_ref[...]

    return pl.pallas_call(
        body,
        out_shape=jax.ShapeDtypeStruct((O, VPAD), jnp.float32),
    )(emb_pad, W, b.reshape(O, 1))


def _sc_lookup(table4, ids_phys):
    O = table4.shape[0]
    mesh = plsc.VectorSubcoreMesh(core_axis_name="c", subcore_axis_name="s")

    @functools.partial(
        pl.kernel,
        out_type=jax.ShapeDtypeStruct((8 * CB, RB, O * 128), jnp.float32),
        mesh=mesh,
        scratch_types=[
            pltpu.VMEM((O, VPAD), jnp.float32),
            [pltpu.VMEM((CCH, GRP), jnp.int32) for _ in range(2)],
            [pltpu.VMEM((8 * CCH, O * 128), jnp.float32) for _ in range(2)],
            [pltpu.SemaphoreType.DMA for _ in range(2)],
            [pltpu.SemaphoreType.DMA for _ in range(2)],
        ],
        compiler_params=pltpu.CompilerParams(
            use_tc_tiling_on_sc=False, needs_layout_passes=False),
    )
    def body(t_ref, ids_ref, out_ref, t_v, in_v, out_v, in_sem, out_sem):
        wid = lax.axis_index("s") * NC + lax.axis_index("c")
        pltpu.sync_copy(t_ref, t_v)
        col_splat = [jnp.full((LANE,), c, jnp.int32) for c in range(O)]

        def coords(it):
            rblk = wid * RPW + it // CCH
            c0 = (it % CCH) * CCH
            return rblk, c0

        def start_in(it, buf):
            rblk, c0 = coords(it)
            pltpu.async_copy(
                ids_ref.at[pl.ds(c0, CCH), rblk], in_v[buf], in_sem[buf])

        def wait_in(buf):
            pltpu.make_async_copy(
                ids_ref.at[pl.ds(0, CCH), 0], in_v[buf], in_sem[buf]).wait()

        def start_out(it, buf):
            rblk, c0 = coords(it)
            pltpu.async_copy(
                out_v[buf], out_ref.at[pl.ds(8 * c0, 8 * CCH), rblk],
                out_sem[buf])

        def wait_out(buf):
            pltpu.make_async_copy(
                out_v[buf], out_ref.at[pl.ds(0, 8 * CCH), 0],
                out_sem[buf]).wait()

        start_in(0, 0)
        start_in(1, 1)

        def step(i, carry):
            for buf in range(2):
                it = 2 * i + buf
                wait_in(buf)

                @pl.when(it >= 2)
                def _():
                    wait_out(buf)

                for ci in range(CCH):
                    for s in range(GRP // LANE):
                        ids16 = in_v[buf][ci, pl.ds(s * LANE, LANE)]
                        row = ci * 8 + s // 8
                        b0 = (s % 8) * LANE
                        for c in range(O):
                            vals = plsc.load_gather(t_v, [col_splat[c], ids16])
                            out_v[buf][row, pl.ds(c * 128 + b0, LANE)] = vals
                start_out(it, buf)

                @pl.when(it + 2 < ITERS)
                def _():
                    start_in(it + 2, buf)

            return carry

        lax.fori_loop(0, ITERS // 2, step, 0)
        wait_out(0)
        wait_out(1)

    return body(table4, ids_phys)


def kernel(input_ids, emb_table, W, b):
    Bsz, Lseq = input_ids.shape
    V, D = emb_table.shape
    O = W.shape[0]
    emb_pad = jnp.zeros((VPAD, D), jnp.float32).at[:V].set(emb_table)
    table4 = _fold_linear(emb_pad, W, b)
    ids_phys = (input_ids.reshape(RB, 128, CB, 8)
                .transpose(2, 0, 3, 1).reshape(CB, RB, GRP))
    out_phys = _sc_lookup(table4, ids_phys)
    out = (out_phys.reshape(Lseq, RB, O, 128)
           .transpose(1, 3, 0, 2).reshape(Bsz, Lseq, O))
    return out


# speedup vs baseline: 64.3302x; 1.0600x over previous
import functools

import jax
import jax.numpy as jnp
from jax import lax
from jax.experimental import pallas as pl
from jax.experimental.pallas import tpu as pltpu
from jax.experimental.pallas import tpu_sc as plsc

NC, NS = 2, 16
NW = NC * NS
LANE = 16
VPAD = 128

CB = 25
RB = 128
GRP = 1024
CCH = 5
RPW = RB // NW
ITERS = RPW * (CB // CCH)


def _fold_linear(emb_pad, W, b):
    O = W.shape[0]

    def body(emb_ref, w_ref, b_ref, t_ref):
        t_ref[...] = lax.dot_general(
            w_ref[...], emb_ref[...], (((1,), (1,)), ((), ())),
            preferred_element_type=jnp.float32) + b_ref[...]

    return pl.pallas_call(
        body,
        out_shape=jax.ShapeDtypeStruct((O, VPAD), jnp.float32),
    )(emb_pad, W, b.reshape(O, 1))


def _sc_lookup(table4, ids_phys):
    O = table4.shape[0]
    mesh = plsc.VectorSubcoreMesh(core_axis_name="c", subcore_axis_name="s")

    @functools.partial(
        pl.kernel,
        out_type=jax.ShapeDtypeStruct((8 * CB, RB, O * 128), jnp.float32),
        mesh=mesh,
        scratch_types=[
            pltpu.VMEM((O, VPAD), jnp.float32),
            pltpu.VMEM((O * VPAD * LANE,), jnp.float32),
            [pltpu.VMEM((CCH, GRP), jnp.int32) for _ in range(2)],
            [pltpu.VMEM((8 * CCH, O * 128), jnp.float32) for _ in range(2)],
            [pltpu.SemaphoreType.DMA for _ in range(2)],
            [pltpu.SemaphoreType.DMA for _ in range(2)],
        ],
        compiler_params=pltpu.CompilerParams(
            use_tc_tiling_on_sc=False, needs_layout_passes=False),
    )
    def body(t_ref, ids_ref, out_ref, t_v, ts_v, in_v, out_v, in_sem, out_sem):
        wid = lax.axis_index("s") * NC + lax.axis_index("c")
        pltpu.sync_copy(t_ref, t_v)

        def spread(s, carry):
            r0 = s * LANE
            for c in range(O):
                vals16 = t_v[c, pl.ds(r0, LANE)]
                for k in range(LANE):
                    ts_v[pl.ds((r0 + k) * (O * LANE) + c * LANE, LANE)] = (
                        jnp.broadcast_to(vals16[k], (LANE,)))
            return carry

        lax.fori_loop(0, VPAD // LANE, spread, 0)
        offs = [lax.iota(jnp.int32, LANE) + c * LANE for c in range(O)]

        def coords(it):
            rblk = wid * RPW + it // CCH
            c0 = (it % CCH) * CCH
            return rblk, c0

        def start_in(it, buf):
            rblk, c0 = coords(it)
            pltpu.async_copy(
                ids_ref.at[pl.ds(c0, CCH), rblk], in_v[buf], in_sem[buf])

        def wait_in(buf):
            pltpu.make_async_copy(
                ids_ref.at[pl.ds(0, CCH), 0], in_v[buf], in_sem[buf]).wait()

        def start_out(it, buf):
            rblk, c0 = coords(it)
            pltpu.async_copy(
                out_v[buf], out_ref.at[pl.ds(8 * c0, 8 * CCH), rblk],
                out_sem[buf])

        def wait_out(buf):
            pltpu.make_async_copy(
                out_v[buf], out_ref.at[pl.ds(0, 8 * CCH), 0],
                out_sem[buf]).wait()

        start_in(0, 0)
        start_in(1, 1)

        def step(i, carry):
            for buf in range(2):
                it = 2 * i + buf
                wait_in(buf)

                @pl.when(it >= 2)
                def _():
                    wait_out(buf)

                for ci in range(CCH):
                    for s in range(GRP // LANE):
                        ids16 = in_v[buf][ci, pl.ds(s * LANE, LANE)]
                        base = ids16 * (O * LANE)
                        row = ci * 8 + s // 8
                        b0 = (s % 8) * LANE
                        for c in range(O):
                            vals = plsc.load_gather(ts_v, [base + offs[c]])
                            out_v[buf][row, pl.ds(c * 128 + b0, LANE)] = vals
                start_out(it, buf)

                @pl.when(it + 2 < ITERS)
                def _():
                    start_in(it + 2, buf)

            return carry

        lax.fori_loop(0, ITERS // 2, step, 0)
        wait_out(0)
        wait_out(1)

    return body(table4, ids_phys)


def kernel(input_ids, emb_table, W, b):
    Bsz, Lseq = input_ids.shape
    V, D = emb_table.shape
    O = W.shape[0]
    emb_pad = jnp.zeros((VPAD, D), jnp.float32).at[:V].set(emb_table)
    table4 = _fold_linear(emb_pad, W, b)
    ids_phys = (input_ids.reshape(RB, 128, CB, 8)
                .transpose(2, 0, 3, 1).reshape(CB, RB, GRP))
    out_phys = _sc_lookup(table4, ids_phys)
    out = (out_phys.reshape(Lseq, RB, O, 128)
           .transpose(1, 3, 0, 2).reshape(Bsz, Lseq, O))
    return out


# speedup vs baseline: 251.6518x; 3.9119x over previous
import functools

import jax
import jax.numpy as jnp
from jax import lax
from jax.experimental import pallas as pl
from jax.experimental.pallas import tpu as pltpu
from jax.experimental.pallas import tpu_sc as plsc

NC, NS = 2, 16
NW = NC * NS
LANE = 16
VPAD = 128

CB = 25
RB = 128
GRP = 1024
CCH = 5
RPW = RB // NW
ITERS = RPW * (CB // CCH)


def _fold_linear(emb_pad, W, b):
    O = W.shape[0]

    def body(emb_ref, w_ref, b_ref, t_ref):
        t_ref[...] = lax.dot_general(
            w_ref[...], emb_ref[...], (((1,), (1,)), ((), ())),
            preferred_element_type=jnp.float32) + b_ref[...]

    return pl.pallas_call(
        body,
        out_shape=jax.ShapeDtypeStruct((O, VPAD), jnp.float32),
    )(emb_pad, W, b.reshape(O, 1))


def _sc_lookup(table4, ids_phys):
    O = table4.shape[0]
    mesh = plsc.VectorSubcoreMesh(core_axis_name="c", subcore_axis_name="s")

    @functools.partial(
        pl.kernel,
        out_type=jax.ShapeDtypeStruct((8 * CB, RB, O * 128), jnp.float32),
        mesh=mesh,
        scratch_types=[
            pltpu.VMEM((O, VPAD), jnp.float32),
            pltpu.VMEM((O * VPAD * LANE,), jnp.float32),
            [pltpu.VMEM((CCH, GRP), jnp.int32) for _ in range(2)],
            [pltpu.VMEM((8 * CCH, O * 128), jnp.float32) for _ in range(2)],
            [pltpu.SemaphoreType.DMA for _ in range(2)],
            [pltpu.SemaphoreType.DMA for _ in range(2)],
        ],
        compiler_params=pltpu.CompilerParams(
            use_tc_tiling_on_sc=False, needs_layout_passes=False),
    )
    def body(t_ref, ids_ref, out_ref, t_v, ts_v, in_v, out_v, in_sem, out_sem):
        wid = lax.axis_index("s") * NC + lax.axis_index("c")
        pltpu.sync_copy(t_ref, t_v)

        def spread(s, carry):
            r0 = s * LANE
            for c in range(O):
                vals16 = t_v[c, pl.ds(r0, LANE)]
                for k in range(LANE):
                    ts_v[pl.ds((r0 + k) * (O * LANE) + c * LANE, LANE)] = (
                        jnp.broadcast_to(vals16[k], (LANE,)))
            return carry

        lax.fori_loop(0, VPAD // LANE, spread, 0)
        offs = [lax.iota(jnp.int32, LANE) + c * LANE for c in range(O)]

        def coords(it):
            rblk = wid * RPW + it // CCH
            c0 = (it % CCH) * CCH
            return rblk, c0

        def start_in(it, buf):
            rblk, c0 = coords(it)
            pltpu.async_copy(
                ids_ref.at[pl.ds(c0, CCH), rblk], in_v[buf], in_sem[buf])

        def wait_in(buf):
            pltpu.make_async_copy(
                ids_ref.at[pl.ds(0, CCH), 0], in_v[buf], in_sem[buf]).wait()

        def start_out(it, buf):
            rblk, c0 = coords(it)
            pltpu.async_copy(
                out_v[buf], out_ref.at[pl.ds(8 * c0, 8 * CCH), rblk],
                out_sem[buf])

        def wait_out(buf):
            pltpu.make_async_copy(
                out_v[buf], out_ref.at[pl.ds(0, 8 * CCH), 0],
                out_sem[buf]).wait()

        start_in(0, 0)
        start_in(1, 1)

        def step(i, carry):
            for buf in range(2):
                it = 2 * i + buf
                wait_in(buf)

                @pl.when(it >= 2)
                def _():
                    wait_out(buf)

                for ci in range(CCH):
                    @plsc.parallel_loop(0, GRP // LANE, unroll=8)
                    def _(s, _ci=ci, _buf=buf):
                        ids16 = in_v[_buf][_ci, pl.ds(s * LANE, LANE)]
                        base = ids16 * (O * LANE)
                        row = _ci * 8 + s // 8
                        b0 = (s % 8) * LANE
                        for c in range(O):
                            vals = plsc.load_gather(ts_v, [base + offs[c]])
                            out_v[_buf][row, pl.ds(c * 128 + b0, LANE)] = vals
                start_out(it, buf)

                @pl.when(it + 2 < ITERS)
                def _():
                    start_in(it + 2, buf)

            return carry

        lax.fori_loop(0, ITERS // 2, step, 0)
        wait_out(0)
        wait_out(1)

    return body(table4, ids_phys)


def kernel(input_ids, emb_table, W, b):
    Bsz, Lseq = input_ids.shape
    V, D = emb_table.shape
    O = W.shape[0]
    emb_pad = jnp.zeros((VPAD, D), jnp.float32).at[:V].set(emb_table)
    table4 = _fold_linear(emb_pad, W, b)
    ids_phys = (input_ids.reshape(RB, 128, CB, 8)
                .transpose(2, 0, 3, 1).reshape(CB, RB, GRP))
    out_phys = _sc_lookup(table4, ids_phys)
    out = (out_phys.reshape(Lseq, RB, O, 128)
           .transpose(1, 3, 0, 2).reshape(Bsz, Lseq, O))
    return out
